# Initial kernel scaffold; baseline (speedup 1.0000x reference)
import jax
import jax.numpy as jnp
from jax.experimental import pallas as pl


def kernel(gene_feat, train_feat, edge_index, W_self1, W_neigh1, b1, W_self2, W_neigh2, b2, Wc1, bc1, Wc2, bc2):
    raise NotImplementedError("write your pallas kernel here")


# speedup vs baseline: 5.7994x; 5.7994x over previous
import functools

import jax
import jax.numpy as jnp
from jax import lax
from jax.experimental import pallas as pl
from jax.experimental.pallas import tpu as pltpu
from jax.experimental.pallas import tpu_sc as plsc

N_GENE = 10000
N_TRAIN = 10000
E = 320000
D = 128
AUG = 144

SC_CORES = 2
SC_SUBCORES = 16
NW = SC_CORES * SC_SUBCORES
CHUNK = 128
NCHUNK = E // CHUNK

N_PAD = 10240
ROWS_PER_TILE = N_PAD // SC_SUBCORES

R_BLK = 1024


def _sc_segment_sum(gene_aug, src, dst, zeros_init):
    mesh = plsc.VectorSubcoreMesh(core_axis_name="c", subcore_axis_name="s")

    @functools.partial(
        pl.kernel,
        out_type=jax.ShapeDtypeStruct((SC_CORES, N_PAD, AUG), jnp.float32),
        mesh=mesh,
        scratch_types=[
            pltpu.VMEM((CHUNK,), jnp.int32),
            pltpu.VMEM((CHUNK,), jnp.int32),
            pltpu.VMEM((CHUNK, AUG), jnp.float32),
            pltpu.VMEM_SHARED((N_PAD, AUG), jnp.float32),
            pltpu.SemaphoreType.DMA,
        ],
        compiler_params=pltpu.CompilerParams(use_tc_tiling_on_sc=False),
    )
    def seg_kernel(gene_hbm, src_hbm, dst_hbm, zero_hbm, out_hbm,
                   src_v, dst_v, rows_v, accum, sem):
        cid = lax.axis_index("c")
        sid = lax.axis_index("s")
        wid = sid * SC_CORES + cid
        row0 = pl.multiple_of(sid * ROWS_PER_TILE, 8)

        pltpu.sync_copy(zero_hbm, accum.at[pl.ds(row0, ROWS_PER_TILE)])
        plsc.subcore_barrier()

        nch = jnp.where(wid < NCHUNK % NW, NCHUNK // NW + 1, NCHUNK // NW)

        def body(i, carry):
            base = pl.multiple_of((wid + i * NW) * CHUNK, 8)
            pltpu.sync_copy(src_hbm.at[pl.ds(base, CHUNK)], src_v)
            pltpu.sync_copy(dst_hbm.at[pl.ds(base, CHUNK)], dst_v)
            pltpu.async_copy(gene_hbm.at[src_v], rows_v, sem).wait()
            pltpu.sync_copy(rows_v, accum.at[dst_v], add=True)
            return carry

        lax.fori_loop(0, nch, body, 0)
        plsc.subcore_barrier()

        pltpu.sync_copy(
            accum.at[pl.ds(row0, ROWS_PER_TILE)],
            out_hbm.at[cid, pl.ds(row0, ROWS_PER_TILE)],
        )

    return seg_kernel(gene_aug, src, dst, zeros_init)


def _tc_dense(p0, p1, train_feat, W_self1, W_neigh1, b1, W_self2, W_neigh2,
              b2, Wc1, bc1, Wc2, bc2):
    nc = Wc2.shape[-1]

    def body(p0_ref, p1_ref, t_ref, ws1, wn1, b1r, ws2, wn2, b2r, wc1, bc1r,
             wc2, bc2r, o_ref):
        p = p0_ref[...] + p1_ref[...]
        deg = jnp.maximum(p[:, D:D + 1], 1.0)
        neigh = p[:, :D] / deg
        h1 = t_ref[...] @ ws1[...] + neigh @ wn1[...] + b1r[...]
        h1 = jnp.maximum(h1, 0.0)
        h2 = h1 @ ws2[...] + neigh @ wn2[...] + b2r[...]
        h2 = jnp.maximum(h2, 0.0)
        h3 = jnp.maximum(h2 @ wc1[...] + bc1r[...], 0.0)
        o_ref[...] = h3 @ wc2[...] + bc2r[...]

    grid = (N_PAD // R_BLK,)
    row_spec = lambda w: pl.BlockSpec((R_BLK, w), lambda i: (i, 0))
    full_spec = lambda a: pl.BlockSpec(a.shape, lambda i: (0,) * a.ndim)

    return pl.pallas_call(
        body,
        grid=grid,
        in_specs=[
            row_spec(AUG), row_spec(AUG), row_spec(D),
            full_spec(W_self1), full_spec(W_neigh1), full_spec(b1),
            full_spec(W_self2), full_spec(W_neigh2), full_spec(b2),
            full_spec(Wc1), full_spec(bc1), full_spec(Wc2), full_spec(bc2),
        ],
        out_specs=pl.BlockSpec((R_BLK, nc), lambda i: (i, 0)),
        out_shape=jax.ShapeDtypeStruct((N_PAD, nc), jnp.float32),
    )(p0, p1, train_feat, W_self1, W_neigh1, b1, W_self2, W_neigh2, b2,
      Wc1, bc1, Wc2, bc2)


def kernel(gene_feat, train_feat, edge_index, W_self1, W_neigh1, b1,
           W_self2, W_neigh2, b2, Wc1, bc1, Wc2, bc2):
    src = edge_index[0].astype(jnp.int32)
    dst = edge_index[1].astype(jnp.int32)

    gene_aug = jnp.concatenate(
        [gene_feat,
         jnp.ones((N_GENE, 1), jnp.float32),
         jnp.zeros((N_GENE, AUG - D - 1), jnp.float32)], axis=1)
    zeros_init = jnp.zeros((ROWS_PER_TILE, AUG), jnp.float32)

    partials = _sc_segment_sum(gene_aug, src, dst, zeros_init)

    train_pad = jnp.pad(train_feat, ((0, N_PAD - N_TRAIN), (0, 0)))
    out = _tc_dense(
        partials[0], partials[1], train_pad,
        W_self1, W_neigh1, b1.reshape(1, -1),
        W_self2, W_neigh2, b2.reshape(1, -1),
        Wc1, bc1.reshape(1, -1), Wc2, bc2.reshape(1, -1))
    return out[:N_TRAIN]
